# baseline (speedup 1.0000x reference)
import functools

import jax
import jax.numpy as jnp
from jax import lax
from jax.experimental import pallas as pl
from jax.experimental.pallas import tpu as pltpu
from jax.experimental.pallas import tpu_sc as plsc

NC = 2
NS = 16
NW = NC * NS
P_CHUNK = 4


def _emb_kernel(L, B, V, D):
    R = L * B
    rows_per_w = R // NW
    pos_per_w = L // NW
    chunks = pos_per_w // P_CHUNK
    rows_chunk = B * P_CHUNK

    mesh = plsc.VectorSubcoreMesh(core_axis_name="c", subcore_axis_name="s")

    @functools.partial(
        pl.kernel,
        mesh=mesh,
        out_type=jax.ShapeDtypeStruct((L, B, D), jnp.float32),
        scratch_types=[
            pltpu.VMEM((rows_per_w,), jnp.int32),
            pltpu.VMEM((V * D,), jnp.float32),
            [pltpu.VMEM((P_CHUNK, B, D), jnp.float32) for _ in range(2)],
            [pltpu.VMEM((P_CHUNK, D), jnp.float32) for _ in range(2)],
            [pltpu.SemaphoreType.DMA for _ in range(4)],
        ],
    )
    def run(x_hbm, w_hbm, pe_hbm, out_hbm, idx_v, w_v, obuf, pebuf, sems):
        psem = sems[0:2]
        osem = sems[2:4]
        sid = lax.axis_index("s")
        wid = sid * NC + lax.axis_index("c")
        row_base = wid * rows_per_w
        pos_base = wid * pos_per_w
        pltpu.sync_copy(x_hbm.at[pl.ds(row_base, rows_per_w)], idx_v)
        pltpu.sync_copy(w_hbm, w_v)
        lane = lax.iota(jnp.int32, 16)

        def issue_pe(gg, b):
            pltpu.async_copy(
                pe_hbm.at[pl.ds(pos_base + gg * P_CHUNK, P_CHUNK)],
                pebuf[b], psem[b])

        issue_pe(0, 0)
        issue_pe(1, 1)

        def slot(gg, b):
            pos0 = pos_base + gg * P_CHUNK
            pltpu.make_async_copy(
                pe_hbm.at[pl.ds(pos0, P_CHUNK)], pebuf[b], psem[b]).wait()
            @pl.when(gg >= 2)
            def _():
                pltpu.make_async_copy(
                    obuf[b],
                    out_hbm.at[pl.ds(pos0 - 2 * P_CHUNK, P_CHUNK)],
                    osem[b]).wait()

            ids16 = idx_v[pl.ds(gg * rows_chunk, rows_chunk)]
            for p in range(P_CHUNK):
                base = [ids16[B * p + bb] * D for bb in range(B)]

                @plsc.parallel_loop(0, D // 16, unroll=8)
                def add_body(c):
                    pev = pebuf[b][p, pl.ds(c * 16, 16)]
                    for bb in range(B):
                        wv = w_v[pl.ds(base[bb] + c * 16, 16)]
                        obuf[b][p, bb, pl.ds(c * 16, 16)] = wv + pev
            pltpu.async_copy(obuf[b], out_hbm.at[pl.ds(pos0, P_CHUNK)],
                             osem[b])

            @pl.when(gg + 2 < chunks)
            def _():
                issue_pe(gg + 2, b)

        def pair_body(i, carry):
            slot(2 * i, 0)
            slot(2 * i + 1, 1)
            return carry

        lax.fori_loop(0, chunks // 2, pair_body, 0)

        for b in range(2):
            gg = chunks - 2 + b
            pltpu.make_async_copy(
                obuf[b],
                out_hbm.at[pl.ds(pos_base + gg * P_CHUNK, P_CHUNK)],
                osem[b]).wait()

    return run


def kernel(x, W_emb, pe):
    L, B = x.shape
    V, D = W_emb.shape
    x_flat = x.reshape(L * B)
    w_flat = W_emb.reshape(V * D)
    pe_flat = pe.reshape(pe.shape[0], D)
    return _emb_kernel(L, B, V, D)(x_flat, w_flat, pe_flat)
